# baseline (device time: 64766 ns/iter reference)
import jax
import jax.numpy as jnp
from jax import lax
from jax.experimental import pallas as pl
from jax.experimental.pallas import tpu as pltpu

N_DEV = 4
SQ = 512
D = 1024
H = 8
DH = 128
SCALE = 0.08838834764831843


def kernel(x, Wq, Wo, Wk, Wv):
    def body(
        x_hbm, wq_hbm, wo_hbm, wk_hbm, wv_hbm, out_ref,
        x_ref, wq_ref, wo_ref, wk_ref, wv_ref,
        xg_ref, wqb_ref, wkb_ref, wvb_ref, wob_ref, o_scr,
        p_send_ref, p_recv_ref, acc_ref,
        in_sems, x_send_sems, x_recv_sems, p_send_sems, p_recv_sems,
    ):
        my = lax.axis_index("i")

        in_copies = []
        for i, (src, dst) in enumerate((
            (x_hbm, x_ref), (wq_hbm, wq_ref), (wk_hbm, wk_ref),
            (wv_hbm, wv_ref), (wo_hbm, wo_ref),
        )):
            cp = pltpu.make_async_copy(src, dst, in_sems.at[i])
            cp.start()
            in_copies.append(cp)

        barrier = pltpu.get_barrier_semaphore()
        for s in (1, N_DEV - 1):
            pl.semaphore_signal(
                barrier, inc=1,
                device_id=((my + s) % N_DEV,),
                device_id_type=pl.DeviceIdType.MESH,
            )
        pl.semaphore_wait(barrier, 2)

        right = (my + 1) % N_DEV
        left = (my + 3) % N_DEV
        opp = (my + 2) % N_DEV
        in_copies[0].wait()
        xg_ref[my] = x_ref[0].astype(jnp.bfloat16)
        x_rdmas = []
        for j, tgt in ((0, right), (1, left)):
            rd = pltpu.make_async_remote_copy(
                src_ref=xg_ref.at[my],
                dst_ref=xg_ref.at[my],
                send_sem=x_send_sems.at[j],
                recv_sem=x_recv_sems.at[my, 0],
                device_id=(tgt,),
                device_id_type=pl.DeviceIdType.MESH,
            )
            rd.start()
            x_rdmas.append(rd)

        in_copies[1].wait()
        wqb_ref[...] = (wq_ref[...] * SCALE).astype(jnp.bfloat16)
        in_copies[2].wait()
        wkb_ref[...] = wk_ref[...].astype(jnp.bfloat16)
        in_copies[3].wait()
        wvb_ref[...] = wv_ref[...].astype(jnp.bfloat16)
        in_copies[4].wait()
        wob_ref[...] = wo_ref[...].astype(jnp.bfloat16)

        def qkv(slot):
            xb = xg_ref[slot]
            q = jnp.dot(
                xb, wqb_ref[...], preferred_element_type=jnp.float32
            ).astype(jnp.bfloat16)
            k = jnp.dot(
                xb, wkb_ref[...], preferred_element_type=jnp.float32
            ).astype(jnp.bfloat16)
            v = jnp.dot(
                xb, wvb_ref[...], preferred_element_type=jnp.float32
            ).astype(jnp.bfloat16)
            return q, k, v

        def attn_proj(q, k, v, r0, nr):
            rows = slice(r0, r0 + nr)
            for h in range(H):
                sl = slice(h * DH, (h + 1) * DH)
                sc = lax.dot_general(
                    q[rows, sl], k[:, sl],
                    dimension_numbers=(((1,), (1,)), ((), ())),
                    preferred_element_type=jnp.float32,
                )
                m = jnp.max(sc, axis=1, keepdims=True)
                p = jnp.exp(sc - m)
                l = jnp.sum(p, axis=1, keepdims=True)
                o = jnp.dot(
                    p.astype(jnp.bfloat16), v[:, sl],
                    preferred_element_type=jnp.float32,
                ) / l
                o_scr[rows, sl] = o.astype(jnp.bfloat16)
            return jnp.dot(
                o_scr[rows, :], wob_ref[...], preferred_element_type=jnp.float32
            )

        def compute_batch(slot):
            q, k, v = qkv(slot)
            return attn_proj(q, k, v, 0, SQ)

        HR = SQ // 2

        def x_recv_descriptor(slot, idx, rows):
            return pltpu.make_async_remote_copy(
                src_ref=xg_ref.at[slot, rows],
                dst_ref=xg_ref.at[slot, rows],
                send_sem=x_send_sems.at[0],
                recv_sem=x_recv_sems.at[slot, idx],
                device_id=(slot,),
                device_id_type=pl.DeviceIdType.MESH,
            )

        acc_ref[...] = compute_batch(my)

        x_recv_descriptor(left, 0, pl.ds(0, SQ)).wait_recv()
        x_recv_descriptor(right, 0, pl.ds(0, SQ)).wait_recv()
        for src_slot, idx, rows, tgt in (
            (left, 1, pl.ds(0, HR), right),
            (right, 2, pl.ds(HR, HR), left),
        ):
            fwd = pltpu.make_async_remote_copy(
                src_ref=xg_ref.at[src_slot, rows],
                dst_ref=xg_ref.at[src_slot, rows],
                send_sem=x_send_sems.at[2 if idx == 1 else 3],
                recv_sem=x_recv_sems.at[src_slot, idx],
                device_id=(tgt,),
                device_id_type=pl.DeviceIdType.MESH,
            )
            fwd.start()
            x_rdmas.append(fwd)

        p_rdmas = []
        for j, t in ((0, right), (1, left), (2, opp)):
            if j == 2:
                x_recv_descriptor(opp, 1, pl.ds(0, HR)).wait_recv()
                x_recv_descriptor(opp, 2, pl.ds(HR, HR)).wait_recv()
            qt, kt, vt = qkv(t)
            for half in range(2):
                r0 = half * HR
                p_send_ref[j, r0:r0 + HR, :] = (
                    attn_proj(qt, kt, vt, r0, HR).astype(jnp.bfloat16))
                rd = pltpu.make_async_remote_copy(
                    src_ref=p_send_ref.at[j, pl.ds(r0, HR)],
                    dst_ref=p_recv_ref.at[my, pl.ds(r0, HR)],
                    send_sem=p_send_sems.at[2 * j + half],
                    recv_sem=p_recv_sems.at[my, half],
                    device_id=(t,),
                    device_id_type=pl.DeviceIdType.MESH,
                )
                rd.start()
                p_rdmas.append(rd)

        def p_recv_descriptor(t, half, rows):
            return pltpu.make_async_remote_copy(
                src_ref=p_recv_ref.at[t, rows],
                dst_ref=p_recv_ref.at[t, rows],
                send_sem=p_send_sems.at[0],
                recv_sem=p_recv_sems.at[t, half],
                device_id=(t,),
                device_id_type=pl.DeviceIdType.MESH,
            )

        rows_a, rows_b = pl.ds(0, HR), pl.ds(HR, HR)
        for t in (left, right):
            for half, rows in ((0, rows_a), (1, rows_b)):
                p_recv_descriptor(t, half, rows).wait_recv()
                acc_ref[rows, :] = (
                    acc_ref[rows, :] + p_recv_ref[t, rows].astype(jnp.float32))
        for half, rows in ((0, rows_a), (1, rows_b)):
            p_recv_descriptor(opp, half, rows).wait_recv()
            out_ref[0, rows, :] = (
                acc_ref[rows, :] + p_recv_ref[opp, rows].astype(jnp.float32)
            ).astype(jnp.bfloat16)

        for rd in x_rdmas + p_rdmas:
            rd.wait_send()

    return pl.pallas_call(
        body,
        out_shape=jax.ShapeDtypeStruct((1, SQ, D), jnp.bfloat16),
        in_specs=[pl.BlockSpec(memory_space=pl.ANY)] * 5,
        out_specs=pl.BlockSpec(memory_space=pltpu.VMEM),
        scratch_shapes=[
            pltpu.VMEM((1, SQ, D), jnp.float32),
            pltpu.VMEM((D, D), jnp.float32),
            pltpu.VMEM((D, D), jnp.float32),
            pltpu.VMEM((D, D), jnp.float32),
            pltpu.VMEM((D, D), jnp.float32),
            pltpu.VMEM((N_DEV, SQ, D), jnp.bfloat16),
            pltpu.VMEM((D, D), jnp.bfloat16),
            pltpu.VMEM((D, D), jnp.bfloat16),
            pltpu.VMEM((D, D), jnp.bfloat16),
            pltpu.VMEM((D, D), jnp.bfloat16),
            pltpu.VMEM((SQ, D), jnp.bfloat16),
            pltpu.VMEM((N_DEV - 1, SQ, D), jnp.bfloat16),
            pltpu.VMEM((N_DEV, SQ, D), jnp.bfloat16),
            pltpu.VMEM((SQ, D), jnp.float32),
            pltpu.SemaphoreType.DMA((5,)),
            pltpu.SemaphoreType.DMA((4,)),
            pltpu.SemaphoreType.DMA((N_DEV, 3)),
            pltpu.SemaphoreType.DMA((6,)),
            pltpu.SemaphoreType.DMA((N_DEV, 2)),
        ],
        compiler_params=pltpu.CompilerParams(
            collective_id=0, vmem_limit_bytes=60 * 2**20
        ),
    )(x, Wq, Wo, Wk, Wv)


# device time: 60669 ns/iter; 1.0675x vs baseline; 1.0675x over previous
import jax
import jax.numpy as jnp
from jax import lax
from jax.experimental import pallas as pl
from jax.experimental.pallas import tpu as pltpu

N_DEV = 4
SQ = 512
D = 1024
H = 8
DH = 128
SCALE = 0.08838834764831843


def kernel(x, Wq, Wo, Wk, Wv):
    def body(
        x_hbm, wq_hbm, wo_hbm, wk_hbm, wv_hbm, out_ref,
        x_ref, wq_ref, wo_ref, wk_ref, wv_ref,
        xg_ref, wqb_ref, wkb_ref, wvb_ref, wob_ref, o_scr,
        p_send_ref, p_recv_ref, acc_ref,
        in_sems, x_send_sems, x_recv_sems, p_send_sems, p_recv_sems,
    ):
        my = lax.axis_index("i")

        in_copies = []
        for i, (src, dst) in enumerate((
            (x_hbm, x_ref), (wq_hbm, wq_ref), (wk_hbm, wk_ref),
            (wv_hbm, wv_ref), (wo_hbm, wo_ref),
        )):
            cp = pltpu.make_async_copy(src, dst, in_sems.at[i])
            cp.start()
            in_copies.append(cp)

        barrier = pltpu.get_barrier_semaphore()
        for s in (1, N_DEV - 1):
            pl.semaphore_signal(
                barrier, inc=1,
                device_id=((my + s) % N_DEV,),
                device_id_type=pl.DeviceIdType.MESH,
            )
        pl.semaphore_wait(barrier, 2)

        right = (my + 1) % N_DEV
        left = (my + 3) % N_DEV
        opp = (my + 2) % N_DEV
        in_copies[0].wait()
        xg_ref[my] = x_ref[0].astype(jnp.bfloat16)
        x_rdmas = []
        for j, tgt in ((0, right), (1, left)):
            rd = pltpu.make_async_remote_copy(
                src_ref=xg_ref.at[my],
                dst_ref=xg_ref.at[my],
                send_sem=x_send_sems.at[j],
                recv_sem=x_recv_sems.at[my, 0],
                device_id=(tgt,),
                device_id_type=pl.DeviceIdType.MESH,
            )
            rd.start()
            x_rdmas.append(rd)

        in_copies[1].wait()
        wqb_ref[...] = (wq_ref[...] * SCALE).astype(jnp.bfloat16)
        in_copies[2].wait()
        wkb_ref[...] = wk_ref[...].astype(jnp.bfloat16)
        in_copies[3].wait()
        wvb_ref[...] = wv_ref[...].astype(jnp.bfloat16)
        in_copies[4].wait()
        wob_ref[...] = wo_ref[...].astype(jnp.bfloat16)

        def qkv(slot):
            xb = xg_ref[slot]
            q = jnp.dot(
                xb, wqb_ref[...], preferred_element_type=jnp.float32
            ).astype(jnp.bfloat16)
            k = jnp.dot(
                xb, wkb_ref[...], preferred_element_type=jnp.float32
            ).astype(jnp.bfloat16)
            v = jnp.dot(
                xb, wvb_ref[...], preferred_element_type=jnp.float32
            ).astype(jnp.bfloat16)
            return q, k, v

        def attn_proj(q, k, v, r0, nr):
            rows = slice(r0, r0 + nr)
            for h in range(H):
                sl = slice(h * DH, (h + 1) * DH)
                sc = lax.dot_general(
                    q[rows, sl], k[:, sl],
                    dimension_numbers=(((1,), (1,)), ((), ())),
                    preferred_element_type=jnp.float32,
                )
                m = jnp.max(sc, axis=1, keepdims=True)
                p = jnp.exp(sc - m)
                l = jnp.sum(p, axis=1, keepdims=True)
                o = jnp.dot(
                    p.astype(jnp.bfloat16), v[:, sl],
                    preferred_element_type=jnp.float32,
                ) / l
                o_scr[rows, sl] = o.astype(jnp.bfloat16)
            return jnp.dot(
                o_scr[rows, :], wob_ref[...], preferred_element_type=jnp.float32
            )

        def compute_batch(slot):
            q, k, v = qkv(slot)
            return attn_proj(q, k, v, 0, SQ)

        HR = SQ // 2

        def x_recv_descriptor(slot, idx, rows):
            return pltpu.make_async_remote_copy(
                src_ref=xg_ref.at[slot, rows],
                dst_ref=xg_ref.at[slot, rows],
                send_sem=x_send_sems.at[0],
                recv_sem=x_recv_sems.at[slot, idx],
                device_id=(slot,),
                device_id_type=pl.DeviceIdType.MESH,
            )

        acc_ref[...] = compute_batch(my)

        x_recv_descriptor(left, 0, pl.ds(0, SQ)).wait_recv()
        x_recv_descriptor(right, 0, pl.ds(0, SQ)).wait_recv()
        for src_slot, idx, rows, tgt in (
            (left, 1, pl.ds(0, HR), right),
            (right, 2, pl.ds(HR, HR), left),
        ):
            fwd = pltpu.make_async_remote_copy(
                src_ref=xg_ref.at[src_slot, rows],
                dst_ref=xg_ref.at[src_slot, rows],
                send_sem=x_send_sems.at[2 if idx == 1 else 3],
                recv_sem=x_recv_sems.at[src_slot, idx],
                device_id=(tgt,),
                device_id_type=pl.DeviceIdType.MESH,
            )
            fwd.start()
            x_rdmas.append(fwd)

        p_rdmas = []
        for j, t in ((0, right), (1, left)):
            p_send_ref[j] = compute_batch(t).astype(jnp.bfloat16)
            rd = pltpu.make_async_remote_copy(
                src_ref=p_send_ref.at[j],
                dst_ref=p_recv_ref.at[my],
                send_sem=p_send_sems.at[j],
                recv_sem=p_recv_sems.at[my, 0],
                device_id=(t,),
                device_id_type=pl.DeviceIdType.MESH,
            )
            rd.start()
            p_rdmas.append(rd)

        x_recv_descriptor(opp, 1, pl.ds(0, HR)).wait_recv()
        x_recv_descriptor(opp, 2, pl.ds(HR, HR)).wait_recv()
        q2, k2, v2 = qkv(opp)
        for half in range(2):
            r0 = half * HR
            p_send_ref[2, r0:r0 + HR, :] = (
                attn_proj(q2, k2, v2, r0, HR).astype(jnp.bfloat16))
            rd = pltpu.make_async_remote_copy(
                src_ref=p_send_ref.at[2, pl.ds(r0, HR)],
                dst_ref=p_recv_ref.at[my, pl.ds(r0, HR)],
                send_sem=p_send_sems.at[2 + half],
                recv_sem=p_recv_sems.at[my, half],
                device_id=(opp,),
                device_id_type=pl.DeviceIdType.MESH,
            )
            rd.start()
            p_rdmas.append(rd)

        def p_recv_descriptor(t, half, rows):
            return pltpu.make_async_remote_copy(
                src_ref=p_recv_ref.at[t, rows],
                dst_ref=p_recv_ref.at[t, rows],
                send_sem=p_send_sems.at[0],
                recv_sem=p_recv_sems.at[t, half],
                device_id=(t,),
                device_id_type=pl.DeviceIdType.MESH,
            )

        rows_a, rows_b = pl.ds(0, HR), pl.ds(HR, HR)
        for t in (left, right):
            p_recv_descriptor(t, 0, pl.ds(0, SQ)).wait_recv()
            acc_ref[...] = acc_ref[...] + p_recv_ref[t].astype(jnp.float32)
        for half, rows in ((0, rows_a), (1, rows_b)):
            p_recv_descriptor(opp, half, rows).wait_recv()
            out_ref[0, rows, :] = (
                acc_ref[rows, :] + p_recv_ref[opp, rows].astype(jnp.float32)
            ).astype(jnp.bfloat16)

        for rd in x_rdmas + p_rdmas:
            rd.wait_send()

    return pl.pallas_call(
        body,
        out_shape=jax.ShapeDtypeStruct((1, SQ, D), jnp.bfloat16),
        in_specs=[pl.BlockSpec(memory_space=pl.ANY)] * 5,
        out_specs=pl.BlockSpec(memory_space=pltpu.VMEM),
        scratch_shapes=[
            pltpu.VMEM((1, SQ, D), jnp.float32),
            pltpu.VMEM((D, D), jnp.float32),
            pltpu.VMEM((D, D), jnp.float32),
            pltpu.VMEM((D, D), jnp.float32),
            pltpu.VMEM((D, D), jnp.float32),
            pltpu.VMEM((N_DEV, SQ, D), jnp.bfloat16),
            pltpu.VMEM((D, D), jnp.bfloat16),
            pltpu.VMEM((D, D), jnp.bfloat16),
            pltpu.VMEM((D, D), jnp.bfloat16),
            pltpu.VMEM((D, D), jnp.bfloat16),
            pltpu.VMEM((SQ, D), jnp.bfloat16),
            pltpu.VMEM((N_DEV - 1, SQ, D), jnp.bfloat16),
            pltpu.VMEM((N_DEV, SQ, D), jnp.bfloat16),
            pltpu.VMEM((SQ, D), jnp.float32),
            pltpu.SemaphoreType.DMA((5,)),
            pltpu.SemaphoreType.DMA((4,)),
            pltpu.SemaphoreType.DMA((N_DEV, 3)),
            pltpu.SemaphoreType.DMA((6,)),
            pltpu.SemaphoreType.DMA((N_DEV, 2)),
        ],
        compiler_params=pltpu.CompilerParams(
            collective_id=0, vmem_limit_bytes=60 * 2**20
        ),
    )(x, Wq, Wo, Wk, Wv)


# device time: 60457 ns/iter; 1.0713x vs baseline; 1.0035x over previous
import jax
import jax.numpy as jnp
from jax import lax
from jax.experimental import pallas as pl
from jax.experimental.pallas import tpu as pltpu

N_DEV = 4
SQ = 512
D = 1024
H = 8
DH = 128
SCALE = 0.08838834764831843


def kernel(x, Wq, Wo, Wk, Wv):
    def body(
        x_hbm, wq_hbm, wo_hbm, wk_hbm, wv_hbm, out_hbm,
        x_ref, wq_ref, wo_ref, wk_ref, wv_ref,
        xg_ref, wqb_ref, wkb_ref, wvb_ref, wob_ref, o_scr,
        p_send_ref, p_recv_ref, acc_ref, outb_ref,
        in_sems, out_sems, x_send_sems, x_recv_sems, p_send_sems,
        p_recv_sems,
    ):
        my = lax.axis_index("i")

        in_copies = []
        for i, (src, dst) in enumerate((
            (x_hbm, x_ref), (wq_hbm, wq_ref), (wk_hbm, wk_ref),
            (wv_hbm, wv_ref), (wo_hbm, wo_ref),
        )):
            cp = pltpu.make_async_copy(src, dst, in_sems.at[i])
            cp.start()
            in_copies.append(cp)

        barrier = pltpu.get_barrier_semaphore()
        for s in (1, N_DEV - 1):
            pl.semaphore_signal(
                barrier, inc=1,
                device_id=((my + s) % N_DEV,),
                device_id_type=pl.DeviceIdType.MESH,
            )
        pl.semaphore_wait(barrier, 2)

        right = (my + 1) % N_DEV
        left = (my + 3) % N_DEV
        opp = (my + 2) % N_DEV
        in_copies[0].wait()
        xg_ref[my] = x_ref[0].astype(jnp.bfloat16)
        x_rdmas = []
        for j, tgt in ((0, right), (1, left)):
            rd = pltpu.make_async_remote_copy(
                src_ref=xg_ref.at[my],
                dst_ref=xg_ref.at[my],
                send_sem=x_send_sems.at[j],
                recv_sem=x_recv_sems.at[my, 0],
                device_id=(tgt,),
                device_id_type=pl.DeviceIdType.MESH,
            )
            rd.start()
            x_rdmas.append(rd)

        in_copies[1].wait()
        wqb_ref[...] = (wq_ref[...] * SCALE).astype(jnp.bfloat16)
        in_copies[2].wait()
        wkb_ref[...] = wk_ref[...].astype(jnp.bfloat16)
        in_copies[3].wait()
        wvb_ref[...] = wv_ref[...].astype(jnp.bfloat16)
        in_copies[4].wait()
        wob_ref[...] = wo_ref[...].astype(jnp.bfloat16)

        def qkv(slot):
            xb = xg_ref[slot]
            q = jnp.dot(
                xb, wqb_ref[...], preferred_element_type=jnp.float32
            ).astype(jnp.bfloat16)
            k = jnp.dot(
                xb, wkb_ref[...], preferred_element_type=jnp.float32
            ).astype(jnp.bfloat16)
            v = jnp.dot(
                xb, wvb_ref[...], preferred_element_type=jnp.float32
            ).astype(jnp.bfloat16)
            return q, k, v

        def attn_proj(q, k, v, r0, nr):
            rows = slice(r0, r0 + nr)
            for h in range(H):
                sl = slice(h * DH, (h + 1) * DH)
                sc = lax.dot_general(
                    q[rows, sl], k[:, sl],
                    dimension_numbers=(((1,), (1,)), ((), ())),
                    preferred_element_type=jnp.float32,
                )
                m = jnp.max(sc, axis=1, keepdims=True)
                p = jnp.exp(sc - m)
                l = jnp.sum(p, axis=1, keepdims=True)
                o = jnp.dot(
                    p.astype(jnp.bfloat16), v[:, sl],
                    preferred_element_type=jnp.float32,
                ) / l
                o_scr[rows, sl] = o.astype(jnp.bfloat16)
            return jnp.dot(
                o_scr[rows, :], wob_ref[...], preferred_element_type=jnp.float32
            )

        def compute_batch(slot):
            q, k, v = qkv(slot)
            return attn_proj(q, k, v, 0, SQ)

        HR = SQ // 2

        def x_recv_descriptor(slot, idx, rows):
            return pltpu.make_async_remote_copy(
                src_ref=xg_ref.at[slot, rows],
                dst_ref=xg_ref.at[slot, rows],
                send_sem=x_send_sems.at[0],
                recv_sem=x_recv_sems.at[slot, idx],
                device_id=(slot,),
                device_id_type=pl.DeviceIdType.MESH,
            )

        acc_ref[...] = compute_batch(my)

        x_recv_descriptor(left, 0, pl.ds(0, SQ)).wait_recv()
        x_recv_descriptor(right, 0, pl.ds(0, SQ)).wait_recv()
        for src_slot, idx, rows, tgt in (
            (left, 1, pl.ds(0, HR), right),
            (right, 2, pl.ds(HR, HR), left),
        ):
            fwd = pltpu.make_async_remote_copy(
                src_ref=xg_ref.at[src_slot, rows],
                dst_ref=xg_ref.at[src_slot, rows],
                send_sem=x_send_sems.at[2 if idx == 1 else 3],
                recv_sem=x_recv_sems.at[src_slot, idx],
                device_id=(tgt,),
                device_id_type=pl.DeviceIdType.MESH,
            )
            fwd.start()
            x_rdmas.append(fwd)

        p_rdmas = []
        for j, t in ((0, right), (1, left)):
            p_send_ref[j] = compute_batch(t).astype(jnp.bfloat16)
            rd = pltpu.make_async_remote_copy(
                src_ref=p_send_ref.at[j],
                dst_ref=p_recv_ref.at[my],
                send_sem=p_send_sems.at[j],
                recv_sem=p_recv_sems.at[my, 0],
                device_id=(t,),
                device_id_type=pl.DeviceIdType.MESH,
            )
            rd.start()
            p_rdmas.append(rd)

        x_recv_descriptor(opp, 1, pl.ds(0, HR)).wait_recv()
        x_recv_descriptor(opp, 2, pl.ds(HR, HR)).wait_recv()
        q2, k2, v2 = qkv(opp)
        for half in range(2):
            r0 = half * HR
            p_send_ref[2, r0:r0 + HR, :] = (
                attn_proj(q2, k2, v2, r0, HR).astype(jnp.bfloat16))
            rd = pltpu.make_async_remote_copy(
                src_ref=p_send_ref.at[2, pl.ds(r0, HR)],
                dst_ref=p_recv_ref.at[my, pl.ds(r0, HR)],
                send_sem=p_send_sems.at[2 + half],
                recv_sem=p_recv_sems.at[my, half],
                device_id=(opp,),
                device_id_type=pl.DeviceIdType.MESH,
            )
            rd.start()
            p_rdmas.append(rd)

        def p_recv_descriptor(t, half, rows):
            return pltpu.make_async_remote_copy(
                src_ref=p_recv_ref.at[t, rows],
                dst_ref=p_recv_ref.at[t, rows],
                send_sem=p_send_sems.at[0],
                recv_sem=p_recv_sems.at[t, half],
                device_id=(t,),
                device_id_type=pl.DeviceIdType.MESH,
            )

        rows_a, rows_b = pl.ds(0, HR), pl.ds(HR, HR)
        for t in (left, right):
            p_recv_descriptor(t, 0, pl.ds(0, SQ)).wait_recv()
            acc_ref[...] = acc_ref[...] + p_recv_ref[t].astype(jnp.float32)
        out_copies = []
        for half, rows in ((0, rows_a), (1, rows_b)):
            p_recv_descriptor(opp, half, rows).wait_recv()
            outb_ref[rows, :] = (
                acc_ref[rows, :] + p_recv_ref[opp, rows].astype(jnp.float32)
            ).astype(jnp.bfloat16)
            cp = pltpu.make_async_copy(
                outb_ref.at[rows], out_hbm.at[0, rows], out_sems.at[half]
            )
            cp.start()
            out_copies.append(cp)
        for cp in out_copies:
            cp.wait()

        for rd in x_rdmas + p_rdmas:
            rd.wait_send()

    return pl.pallas_call(
        body,
        out_shape=jax.ShapeDtypeStruct((1, SQ, D), jnp.bfloat16),
        in_specs=[pl.BlockSpec(memory_space=pl.ANY)] * 5,
        out_specs=pl.BlockSpec(memory_space=pl.ANY),
        scratch_shapes=[
            pltpu.VMEM((1, SQ, D), jnp.float32),
            pltpu.VMEM((D, D), jnp.float32),
            pltpu.VMEM((D, D), jnp.float32),
            pltpu.VMEM((D, D), jnp.float32),
            pltpu.VMEM((D, D), jnp.float32),
            pltpu.VMEM((N_DEV, SQ, D), jnp.bfloat16),
            pltpu.VMEM((D, D), jnp.bfloat16),
            pltpu.VMEM((D, D), jnp.bfloat16),
            pltpu.VMEM((D, D), jnp.bfloat16),
            pltpu.VMEM((D, D), jnp.bfloat16),
            pltpu.VMEM((SQ, D), jnp.bfloat16),
            pltpu.VMEM((N_DEV - 1, SQ, D), jnp.bfloat16),
            pltpu.VMEM((N_DEV, SQ, D), jnp.bfloat16),
            pltpu.VMEM((SQ, D), jnp.float32),
            pltpu.VMEM((SQ, D), jnp.bfloat16),
            pltpu.SemaphoreType.DMA((5,)),
            pltpu.SemaphoreType.DMA((2,)),
            pltpu.SemaphoreType.DMA((4,)),
            pltpu.SemaphoreType.DMA((N_DEV, 3)),
            pltpu.SemaphoreType.DMA((6,)),
            pltpu.SemaphoreType.DMA((N_DEV, 2)),
        ],
        compiler_params=pltpu.CompilerParams(
            collective_id=0, vmem_limit_bytes=60 * 2**20
        ),
    )(x, Wq, Wo, Wk, Wv)
